# baseline (speedup 1.0000x reference)
import functools
import math

import jax
import jax.numpy as jnp
from jax import lax
from jax.experimental import pallas as pl
from jax.experimental.pallas import tpu as pltpu
from jax.experimental.pallas import tpu_sc as plsc

_NUM_EXPERTS = 8
_NUM_HASHES = 4
_N_TOKENS = 8192
_BT = 1024
_NCHUNK = 2
_CHUNK = _N_TOKENS // _NCHUNK

_EPS = 1e-8
_LOG_EPS = float(math.log(1e-8))

_NC = 2
_NS = 16
_NW = _NC * _NS
_L = 16


def _tc_body(x_ref, w_ref, hvt_ref):
    x = x_ref[...]
    w = w_ref[...]
    hvt_ref[...] = lax.dot_general(
        w, x, (((1,), (1,)), ((), ())),
        preferred_element_type=jnp.float32)


def _make_sc_route(tpw):
    rpw = tpw * _NUM_EXPERTS // 128

    def _sc_route(hvt_hbm, probs_hbm, logits_hbm, hv_v, probs_v, logits_v):
        wid = lax.axis_index("s") * _NC + lax.axis_index("c")
        tok_base = wid * tpw
        pltpu.sync_copy(hvt_hbm.at[:, pl.ds(tok_base, tpw)], hv_v)

        lane = lax.broadcasted_iota(jnp.int32, (_L,), 0)
        expert_lane = lane % _NUM_EXPERTS
        hi_half = lane >= _NUM_EXPERTS

        def _emit(k, carry):
            h0 = hv_v[0, pl.ds(k * _L, _L)]
            h1 = hv_v[1, pl.ds(k * _L, _L)]
            h2 = hv_v[2, pl.ds(k * _L, _L)]
            h3 = hv_v[3, pl.ds(k * _L, _L)]
            e01 = jnp.where(h1 > h0, 1, 0)
            m01 = jnp.maximum(h0, h1)
            e23 = jnp.where(h3 > h2, 3, 2)
            m23 = jnp.maximum(h2, h3)
            ev16 = jnp.where(m23 > m01, e23, e01) % _NUM_EXPERTS
            for j in range(_L // 2):
                e0 = ev16[2 * j]
                e1 = ev16[2 * j + 1]
                ev = jnp.where(hi_half, e1, e0)
                onehot = expert_lane == ev
                probs_v[k, pl.ds(j * _L, _L)] = jnp.where(
                    onehot, jnp.float32(1.0), jnp.float32(_EPS))
                logits_v[k, pl.ds(j * _L, _L)] = jnp.where(
                    onehot, jnp.float32(0.0), jnp.float32(_LOG_EPS))
            return carry

        lax.fori_loop(0, rpw, _emit, 0)

        pltpu.sync_copy(probs_v, probs_hbm.at[pl.ds(wid * rpw, rpw), :])
        pltpu.sync_copy(logits_v, logits_hbm.at[pl.ds(wid * rpw, rpw), :])

    return _sc_route


def _tc_matmul(x_chunk, hash_weights, d):
    return pl.pallas_call(
        _tc_body,
        grid=(_CHUNK // _BT,),
        in_specs=[
            pl.BlockSpec((_BT, d), lambda i: (i, 0)),
            pl.BlockSpec((_NUM_HASHES, d), lambda i: (0, 0)),
        ],
        out_specs=pl.BlockSpec((_NUM_HASHES, _BT), lambda i: (0, i)),
        out_shape=jax.ShapeDtypeStruct((_NUM_HASHES, _CHUNK), jnp.float32),
        compiler_params=pltpu.CompilerParams(
            dimension_semantics=("arbitrary",)),
    )(x_chunk, hash_weights)


@jax.jit
def kernel(x, hash_weights):
    n, d = x.shape
    tpw = _CHUNK // _NW
    nrows = _CHUNK * _NUM_EXPERTS // 128
    sc_fn = functools.partial(
        pl.kernel,
        mesh=plsc.VectorSubcoreMesh(core_axis_name="c", subcore_axis_name="s"),
        out_type=[
            jax.ShapeDtypeStruct((nrows, 128), jnp.float32),
            jax.ShapeDtypeStruct((nrows, 128), jnp.float32),
        ],
        scratch_types=[
            pltpu.VMEM((_NUM_HASHES, tpw), jnp.float32),
            pltpu.VMEM((nrows // _NW, 128), jnp.float32),
            pltpu.VMEM((nrows // _NW, 128), jnp.float32),
        ],
    )(_make_sc_route(tpw))

    probs_halves, logits_halves = [], []
    for c in range(_NCHUNK):
        hvt_c = _tc_matmul(
            lax.slice_in_dim(x, c * _CHUNK, (c + 1) * _CHUNK, axis=0),
            hash_weights, d)
        p2d, l2d = sc_fn(hvt_c)
        probs_halves.append(p2d.reshape(_CHUNK, _NUM_EXPERTS))
        logits_halves.append(l2d.reshape(_CHUNK, _NUM_EXPERTS))
    probs = jnp.concatenate(probs_halves, axis=0)
    logits = jnp.concatenate(logits_halves, axis=0)
    return (logits, probs)


# speedup vs baseline: 1.8763x; 1.8763x over previous
import functools
import math

import jax
import jax.numpy as jnp
from jax import lax
from jax.experimental import pallas as pl
from jax.experimental.pallas import tpu as pltpu
from jax.experimental.pallas import tpu_sc as plsc

_NUM_EXPERTS = 8
_NUM_HASHES = 4
_N_TOKENS = 8192
_BT = 1024

_EPS = 1e-8
_LOG_EPS = float(math.log(1e-8))

_NC = 2
_NS = 16
_NW = _NC * _NS
_TPW = _N_TOKENS // _NW
_L = 16
_RPW = _TPW * _NUM_EXPERTS // 128


def _tc_body(x_ref, w_ref, hvt_ref):
    x = x_ref[...]
    w = w_ref[...]
    hvt_ref[...] = lax.dot_general(
        w, x, (((1,), (1,)), ((), ())),
        preferred_element_type=jnp.float32)


def _sc_route(hvt_hbm, probs_hbm, logits_hbm, hv_v, probs_v, logits_v):
    wid = lax.axis_index("s") * _NC + lax.axis_index("c")
    tok_base = wid * _TPW
    pltpu.sync_copy(hvt_hbm.at[:, pl.ds(tok_base, _TPW)], hv_v)

    lane = lax.broadcasted_iota(jnp.int32, (_L,), 0)
    expert_lane = lane % _NUM_EXPERTS
    hi_half = lane >= _NUM_EXPERTS

    def _emit(k, carry):
        h0 = hv_v[0, pl.ds(k * _L, _L)]
        h1 = hv_v[1, pl.ds(k * _L, _L)]
        h2 = hv_v[2, pl.ds(k * _L, _L)]
        h3 = hv_v[3, pl.ds(k * _L, _L)]
        e01 = jnp.where(h1 > h0, 1, 0)
        m01 = jnp.maximum(h0, h1)
        e23 = jnp.where(h3 > h2, 3, 2)
        m23 = jnp.maximum(h2, h3)
        ev16 = jnp.where(m23 > m01, e23, e01) % _NUM_EXPERTS
        for j in range(_L // 2):
            e0 = ev16[2 * j]
            e1 = ev16[2 * j + 1]
            ev = jnp.where(hi_half, e1, e0)
            onehot = expert_lane == ev
            probs_v[k, pl.ds(j * _L, _L)] = jnp.where(
                onehot, jnp.float32(1.0), jnp.float32(_EPS))
            logits_v[k, pl.ds(j * _L, _L)] = jnp.where(
                onehot, jnp.float32(0.0), jnp.float32(_LOG_EPS))
        return carry

    lax.fori_loop(0, _RPW, _emit, 0)

    pltpu.sync_copy(probs_v, probs_hbm.at[pl.ds(wid * _RPW, _RPW), :])
    pltpu.sync_copy(logits_v, logits_hbm.at[pl.ds(wid * _RPW, _RPW), :])


@jax.jit
def kernel(x, hash_weights):
    n, d = x.shape
    hvt = pl.pallas_call(
        _tc_body,
        grid=(n // _BT,),
        in_specs=[
            pl.BlockSpec((_BT, d), lambda i: (i, 0)),
            pl.BlockSpec((_NUM_HASHES, d), lambda i: (0, 0)),
        ],
        out_specs=pl.BlockSpec((_NUM_HASHES, _BT), lambda i: (0, i)),
        out_shape=jax.ShapeDtypeStruct((_NUM_HASHES, n), jnp.float32),
        compiler_params=pltpu.CompilerParams(
            dimension_semantics=("arbitrary",)),
    )(x, hash_weights)

    nrows = n * _NUM_EXPERTS // 128
    sc_fn = functools.partial(
        pl.kernel,
        mesh=plsc.VectorSubcoreMesh(core_axis_name="c", subcore_axis_name="s"),
        out_type=[
            jax.ShapeDtypeStruct((nrows, 128), jnp.float32),
            jax.ShapeDtypeStruct((nrows, 128), jnp.float32),
        ],
        scratch_types=[
            pltpu.VMEM((_NUM_HASHES, _TPW), jnp.float32),
            pltpu.VMEM((_RPW, 128), jnp.float32),
            pltpu.VMEM((_RPW, 128), jnp.float32),
        ],
    )(_sc_route)
    probs2d, logits2d = sc_fn(hvt)
    probs = probs2d.reshape(n, _NUM_EXPERTS)
    logits = logits2d.reshape(n, _NUM_EXPERTS)
    return (logits, probs)
